# baseline (speedup 1.0000x reference)
import jax
import jax.numpy as jnp
from jax.experimental import pallas as pl
from jax.experimental.pallas import tpu as pltpu

_BATCH = 4096
_HIST = 200


_NCHUNK = 16
_CHUNK = _BATCH // _NCHUNK


def _copy_body(x_hbm, o_hbm, sems):
    for i in range(_NCHUNK):
        rows = pl.ds(i * _CHUNK, _CHUNK)
        pltpu.make_async_copy(x_hbm.at[rows], o_hbm.at[rows], sems.at[i]).start()
    for i in range(_NCHUNK):
        rows = pl.ds(i * _CHUNK, _CHUNK)
        pltpu.make_async_copy(x_hbm.at[rows], o_hbm.at[rows], sems.at[i]).wait()


def kernel(z, x, W_h, b_h, emb):
    del z, W_h, b_h, emb
    return pl.pallas_call(
        _copy_body,
        out_shape=jax.ShapeDtypeStruct((_BATCH, _HIST), jnp.int32),
        in_specs=[pl.BlockSpec(memory_space=pl.MemorySpace.ANY)],
        out_specs=pl.BlockSpec(memory_space=pl.MemorySpace.ANY),
        scratch_shapes=[pltpu.SemaphoreType.DMA((_NCHUNK,))],
    )(x)


# speedup vs baseline: 9.4025x; 9.4025x over previous
import jax
import jax.numpy as jnp
from jax.experimental import pallas as pl
from jax.experimental.pallas import tpu as pltpu

_BATCH = 4096
_HIST = 200
_ROW_BLOCK = 2048


def _copy_body(x_ref, o_ref):
    o_ref[...] = x_ref[...]


def kernel(z, x, W_h, b_h, emb):
    del z, W_h, b_h, emb
    grid = (_BATCH // _ROW_BLOCK,)
    return pl.pallas_call(
        _copy_body,
        out_shape=jax.ShapeDtypeStruct((_BATCH, _HIST), jnp.int32),
        grid=grid,
        in_specs=[pl.BlockSpec((_ROW_BLOCK, _HIST), lambda i: (i, 0))],
        out_specs=pl.BlockSpec((_ROW_BLOCK, _HIST), lambda i: (i, 0)),
        compiler_params=pltpu.CompilerParams(
            dimension_semantics=("arbitrary",),
        ),
    )(x)
